# baseline (speedup 1.0000x reference)
import functools

import jax
import jax.numpy as jnp
from jax.experimental import pallas as pl
from jax.experimental.pallas import tpu as pltpu

N = 10000
D_IN = 128
D_HID = 32
D_OUT = 16
BR = 400


def _gcn_kernel(x_ref, adj_ref, w1_ref, b1_ref, w2_ref, b2_ref,
                out_ref, s1_ref, h_ref):
    s = pl.program_id(0)
    i = pl.program_id(1)

    @pl.when((s == 0) & (i == 0))
    def _():
        s1_ref[:] = jnp.dot(x_ref[:], w1_ref[:],
                            preferred_element_type=jnp.float32)

    @pl.when(s == 0)
    def _():
        hblk = jnp.dot(adj_ref[:], s1_ref[:],
                       preferred_element_type=jnp.float32) + b1_ref[:]
        h_ref[pl.ds(i * BR, BR), :] = jnp.maximum(hblk, 0.0)
        out_ref[:] = jnp.zeros_like(out_ref)

    @pl.when(s == 1)
    def _():
        t = jnp.dot(adj_ref[:], h_ref[:],
                    preferred_element_type=jnp.float32)
        out_ref[:] = jnp.dot(t, w2_ref[:],
                             preferred_element_type=jnp.float32) + b2_ref[:]


@jax.jit
def kernel(x, adj_norm, W1, b1, W2, b2):
    nr = N // BR
    grid = (2, nr)
    return pl.pallas_call(
        _gcn_kernel,
        grid=grid,
        in_specs=[
            pl.BlockSpec((N, D_IN), lambda s, i: (0, 0)),
            pl.BlockSpec((BR, N), lambda s, i: (i, 0)),
            pl.BlockSpec((D_IN, D_HID), lambda s, i: (0, 0)),
            pl.BlockSpec((1, D_HID), lambda s, i: (0, 0)),
            pl.BlockSpec((D_HID, D_OUT), lambda s, i: (0, 0)),
            pl.BlockSpec((1, D_OUT), lambda s, i: (0, 0)),
        ],
        out_specs=pl.BlockSpec((BR, D_OUT), lambda s, i: (i, 0)),
        out_shape=jax.ShapeDtypeStruct((N, D_OUT), jnp.float32),
        scratch_shapes=[
            pltpu.VMEM((N, D_HID), jnp.float32),
            pltpu.VMEM((N, D_HID), jnp.float32),
        ],
        compiler_params=pltpu.CompilerParams(
            dimension_semantics=("arbitrary", "arbitrary"),
        ),
    )(x, adj_norm, W1, b1.reshape(1, D_HID), W2, b2.reshape(1, D_OUT))
